# baseline (speedup 1.0000x reference)
import jax
import jax.numpy as jnp
from jax import lax
from jax.experimental import pallas as pl
from jax.experimental.pallas import tpu as pltpu
from jax.experimental.pallas import tpu_sc as plsc

B = 64
N_ROI = 347
N_NET = 16
D = 128
H = 128
HALF = 64
NC = 4
N_TOT = B * N_ROI
NN = B * N_NET
E_ROI = N_TOT * 16
E_NET = NN * 16

NUM_CORES = 2
NUM_SUBCORES = 16

CH = 1024
NCHUNK = 22
EPT = CH * NCHUNK
E_PAD = EPT * NUM_SUBCORES
N_ACC = 22528
ZROWS = N_ACC // NUM_SUBCORES
QW = 32
NQ = 4

ENT = E_NET // NUM_SUBCORES
NPT = NN // NUM_SUBCORES


def _sc_mesh():
    return plsc.VectorSubcoreMesh(
        core_axis_name="c", subcore_axis_name="s",
        num_cores=NUM_CORES, num_subcores=NUM_SUBCORES)


_SC_PARAMS = pltpu.CompilerParams(use_tc_tiling_on_sc=False,
                                  needs_layout_passes=False)


def _make_sage_sc(with_deg: bool):
    out_type = [jax.ShapeDtypeStruct((NQ, N_ACC, QW), jnp.float32)]
    if with_deg:
        out_type.append(jax.ShapeDtypeStruct((N_ACC,), jnp.float32))
    scratch = [
        [pltpu.VMEM((CH,), jnp.float32) for _ in range(2)],
        [pltpu.VMEM((CH,), jnp.int32) for _ in range(2)],
        [pltpu.VMEM((CH,), jnp.int32) for _ in range(2)],
        [pltpu.VMEM((CH, QW), jnp.float32) for _ in range(2)],
        pltpu.VMEM_SHARED((N_ACC, QW), jnp.float32),
        [pltpu.SemaphoreType.DMA for _ in range(2)],
    ]
    if with_deg:
        scratch += [
            pltpu.VMEM((CH,), jnp.float32),
            pltpu.VMEM_SHARED((N_ACC,), jnp.float32),
        ]

    def body(xs_hbm, pk_hbm, z2_hbm, z1_hbm, out_hbm, *rest):
        if with_deg:
            deg_hbm, pkv, srcv, dstv, rows, acc, sem, onesv, dacc = rest
        else:
            pkv, srcv, dstv, rows, acc, sem = rest
        c = lax.axis_index("c")
        s = lax.axis_index("s")
        base = s * EPT
        z0 = pl.multiple_of(s * ZROWS, ZROWS)

        if with_deg:
            @pl.when(c == 0)
            def _():
                @pl.loop(0, CH // 16)
                def _(i):
                    onesv[pl.ds(i * 16, 16)] = jnp.full((16,), 1.0, jnp.float32)

        for q in range(2):
            qidx = 2 * c + q
            pltpu.sync_copy(z2_hbm, acc.at[pl.ds(z0, ZROWS)])
            if with_deg and q == 0:
                @pl.when(c == 0)
                def _():
                    pltpu.sync_copy(z1_hbm, dacc.at[pl.ds(z0, ZROWS)])
            plsc.subcore_barrier()

            qoff = jnp.full((16,), 0, jnp.int32) + qidx

            def load_unpack_start(kk, b):
                off = pl.multiple_of(base + kk * CH, CH)
                pltpu.sync_copy(pk_hbm.at[pl.ds(off, CH)], pkv[b])

                @pl.loop(0, CH // 16, unroll=4)
                def _(g):
                    sl = pl.ds(g * 16, 16)
                    p = plsc.bitcast(pkv[b][sl], jnp.int32)
                    srcv[b][sl] = ((p >> 15) << 2) + qoff
                    dstv[b][sl] = p & 32767

                return pltpu.async_copy(xs_hbm.at[srcv[b]], rows[b], sem[b])

            def wait_scatter(b):
                pltpu.make_async_copy(xs_hbm.at[srcv[b]], rows[b], sem[b]).wait()
                pltpu.sync_copy(rows[b], acc.at[dstv[b]], add=True)
                if with_deg and q == 0:
                    @pl.when(c == 0)
                    def _():
                        pltpu.sync_copy(onesv, dacc.at[dstv[b]], add=True)

            load_unpack_start(0, 0)

            @pl.loop(0, NCHUNK // 2)
            def _(i):
                load_unpack_start(2 * i + 1, 1)
                wait_scatter(0)

                @pl.when(i < NCHUNK // 2 - 1)
                def _():
                    load_unpack_start(2 * i + 2, 0)
                wait_scatter(1)

            plsc.subcore_barrier()
            pltpu.sync_copy(acc.at[pl.ds(z0, ZROWS)],
                            out_hbm.at[qidx, pl.ds(z0, ZROWS)])
            if with_deg and q == 0:
                @pl.when(c == 0)
                def _():
                    pltpu.sync_copy(dacc.at[pl.ds(z0, ZROWS)], deg_hbm.at[pl.ds(z0, ZROWS)])

    return pl.kernel(body, out_type=tuple(out_type) if with_deg else out_type[0],
                     mesh=_sc_mesh(), scratch_types=scratch,
                     compiler_params=_SC_PARAMS)


def _make_gat_sc():
    out_type = jax.ShapeDtypeStruct((NN, D), jnp.float32)
    scratch = [
        pltpu.VMEM((ENT,), jnp.int32),
        pltpu.VMEM((ENT,), jnp.int32),
        pltpu.VMEM((ENT,), jnp.int32),
        pltpu.VMEM((NN,), jnp.float32),
        pltpu.VMEM((NN,), jnp.float32),
        pltpu.VMEM((ENT,), jnp.float32),
        pltpu.VMEM((ENT,), jnp.float32),
        pltpu.VMEM((NN,), jnp.float32),
        pltpu.VMEM((ENT, HALF), jnp.float32),
        pltpu.VMEM_SHARED((NN,), jnp.float32),
        pltpu.VMEM_SHARED((NN, HALF), jnp.float32),
        pltpu.SemaphoreType.DMA,
    ]

    def body(hs_hbm, src_hbm, dst_hbm, asv_hbm, adv_hbm,
             z2_hbm, z1_hbm, out_hbm,
             srcv, gsrcv, dstv, asv, adv, exbuf, coefbuf, denv, rows,
             dacc, nacc, sem):
        c = lax.axis_index("c")
        s = lax.axis_index("s")
        base = s * ENT

        pltpu.sync_copy(src_hbm.at[pl.ds(base, ENT)], srcv)
        pltpu.sync_copy(dst_hbm.at[pl.ds(base, ENT)], dstv)
        pltpu.sync_copy(asv_hbm, asv)
        pltpu.sync_copy(adv_hbm, adv)

        coff = jnp.full((16,), 0, jnp.int32) + c

        @pl.loop(0, ENT // 16, unroll=4)
        def _(g):
            sl = pl.ds(g * 16, 16)
            gsrcv[sl] = (srcv[sl] << 1) + coff

        gat = pltpu.async_copy(hs_hbm.at[gsrcv], rows, sem)

        pltpu.sync_copy(z1_hbm, dacc.at[pl.ds(s * NPT, NPT)])
        pltpu.sync_copy(z2_hbm, nacc.at[pl.ds(s * NPT, NPT)])
        plsc.subcore_barrier()

        @pl.loop(0, ENT // 16)
        def _(g):
            sl = pl.ds(g * 16, 16)
            e = plsc.load_gather(asv, [srcv[sl]]) + plsc.load_gather(adv, [dstv[sl]])
            e = jnp.where(e >= 0, e, 0.2 * e)
            exbuf[sl] = jnp.exp(e)

        pltpu.sync_copy(exbuf, dacc.at[dstv], add=True)
        plsc.subcore_barrier()
        pltpu.sync_copy(dacc, denv)

        @pl.loop(0, ENT // 16)
        def _(g):
            sl = pl.ds(g * 16, 16)
            den = plsc.load_gather(denv, [dstv[sl]])
            coefbuf[sl] = exbuf[sl] / (den + 1e-16)

        gat.wait()

        @pl.loop(0, ENT)
        def _(j):
            cf = plsc.load_gather(coefbuf, [jnp.full((16,), 0, jnp.int32) + j])
            for jj in range(HALF // 16):
                sl2 = pl.ds(jj * 16, 16)
                rows[j, sl2] = rows[j, sl2] * cf

        pltpu.sync_copy(rows, nacc.at[dstv], add=True)
        plsc.subcore_barrier()
        n0 = pl.multiple_of(s * NPT, NPT)
        pltpu.sync_copy(nacc.at[pl.ds(n0, NPT)],
                        out_hbm.at[pl.ds(n0, NPT), pl.ds(c * HALF, HALF)])

    return pl.kernel(body, out_type=out_type, mesh=_sc_mesh(),
                     scratch_types=scratch, compiler_params=_SC_PARAMS)


BN = 2776
BPB = BN // N_ROI


def _sage_dense_body(x_ref, agg_ref, deg_ref, wr_ref, wn_ref, b_ref, out_ref):
    inv = 1.0 / jnp.maximum(deg_ref[...], 1.0)
    h = (jnp.dot(x_ref[...], wr_ref[...], preferred_element_type=jnp.float32)
         + b_ref[...])
    for q in range(NQ):
        h = h + jnp.dot(agg_ref[q] * inv, wn_ref[q],
                        preferred_element_type=jnp.float32)
    out_ref[...] = jax.nn.gelu(h)


def _sage_dense(xin, agg, degc, Wr, Wn, brow):
    return pl.pallas_call(
        _sage_dense_body,
        grid=(N_TOT // BN,),
        in_specs=[
            pl.BlockSpec((BN, D), lambda i: (i, 0)),
            pl.BlockSpec((NQ, BN, QW), lambda i: (0, i, 0)),
            pl.BlockSpec((BN, 1), lambda i: (i, 0)),
            pl.BlockSpec((D, H), lambda i: (0, 0)),
            pl.BlockSpec((NQ, QW, H), lambda i: (0, 0, 0)),
            pl.BlockSpec((1, H), lambda i: (0, 0)),
        ],
        out_specs=pl.BlockSpec((BN, D), lambda i: (i, 0)),
        out_shape=jax.ShapeDtypeStruct((N_TOT, D), jnp.float32),
    )(xin, agg, degc, Wr, Wn, brow)


def _sage_pool_body(x_ref, agg_ref, deg_ref, wr_ref, wn_ref, b_ref,
                    mask_ref, logit_ref, wg_ref, as_ref, ad_ref,
                    hp_ref, asv_ref, adv_ref, alpha_ref):
    @pl.when(pl.program_id(0) == 0)
    def _():
        logits = jnp.where(mask_ref[...] == 0, -20.0, logit_ref[...])
        m = jnp.max(logits, axis=0, keepdims=True)
        ex = jnp.exp(logits - m)
        alpha_ref[...] = ex / jnp.sum(ex, axis=0, keepdims=True)

    inv = 1.0 / jnp.maximum(deg_ref[...], 1.0)
    h = (jnp.dot(x_ref[...], wr_ref[...], preferred_element_type=jnp.float32)
         + b_ref[...])
    for q in range(NQ):
        h = h + jnp.dot(agg_ref[q] * inv, wn_ref[q],
                        preferred_element_type=jnp.float32)
    h = jax.nn.gelu(h)
    alpha = alpha_ref[...]
    zs = [lax.dot_general(alpha, h[N_ROI * b:N_ROI * (b + 1), :],
                          (((0,), (0,)), ((), ())),
                          preferred_element_type=jnp.float32)
          for b in range(BPB)]
    z = jnp.concatenate(zs, axis=0)
    hp = jnp.dot(z, wg_ref[...], preferred_element_type=jnp.float32)
    asv_ref[...] = jnp.dot(hp, as_ref[...], preferred_element_type=jnp.float32)
    adv_ref[...] = jnp.dot(hp, ad_ref[...], preferred_element_type=jnp.float32)
    hp_ref[...] = hp


def _sage_dense_pool(xin, agg, degc, Wr, Wn, brow, mask, pool_logits,
                     Wg0, as_col, ad_col):
    zrows = BPB * N_NET
    return pl.pallas_call(
        _sage_pool_body,
        grid=(N_TOT // BN,),
        in_specs=[
            pl.BlockSpec((BN, D), lambda i: (i, 0)),
            pl.BlockSpec((NQ, BN, QW), lambda i: (0, i, 0)),
            pl.BlockSpec((BN, 1), lambda i: (i, 0)),
            pl.BlockSpec((D, H), lambda i: (0, 0)),
            pl.BlockSpec((NQ, QW, H), lambda i: (0, 0, 0)),
            pl.BlockSpec((1, H), lambda i: (0, 0)),
            pl.BlockSpec((N_ROI, N_NET), lambda i: (0, 0)),
            pl.BlockSpec((N_ROI, N_NET), lambda i: (0, 0)),
            pl.BlockSpec((D, H), lambda i: (0, 0)),
            pl.BlockSpec((H, 1), lambda i: (0, 0)),
            pl.BlockSpec((H, 1), lambda i: (0, 0)),
        ],
        out_specs=[
            pl.BlockSpec((zrows, D), lambda i: (i, 0)),
            pl.BlockSpec((zrows, 1), lambda i: (i, 0)),
            pl.BlockSpec((zrows, 1), lambda i: (i, 0)),
        ],
        out_shape=[
            jax.ShapeDtypeStruct((NN, D), jnp.float32),
            jax.ShapeDtypeStruct((NN, 1), jnp.float32),
            jax.ShapeDtypeStruct((NN, 1), jnp.float32),
        ],
        scratch_shapes=[pltpu.VMEM((N_ROI, N_NET), jnp.float32)],
    )(xin, agg, degc, Wr, Wn, brow, mask, pool_logits, Wg0, as_col, ad_col)


def _gat_mid_body(num_ref, b_ref, wg_ref, as_ref, ad_ref,
                  hp_ref, asv_ref, adv_ref):
    g = jax.nn.gelu(num_ref[...] + b_ref[...])
    h = jnp.dot(g, wg_ref[...], preferred_element_type=jnp.float32)
    asv_ref[...] = jnp.dot(h, as_ref[...], preferred_element_type=jnp.float32)
    adv_ref[...] = jnp.dot(h, ad_ref[...], preferred_element_type=jnp.float32)
    hp_ref[...] = h


def _gat_mid(num0, brow, Wg, as_col, ad_col):
    full = lambda s: pl.BlockSpec(s, lambda: tuple(0 for _ in s))
    return pl.pallas_call(
        _gat_mid_body,
        in_specs=[
            full((NN, D)),
            full((1, H)),
            full((D, H)),
            full((D, 1)),
            full((D, 1)),
        ],
        out_specs=[
            full((NN, D)),
            full((NN, 1)),
            full((NN, 1)),
        ],
        out_shape=[
            jax.ShapeDtypeStruct((NN, D), jnp.float32),
            jax.ShapeDtypeStruct((NN, 1), jnp.float32),
            jax.ShapeDtypeStruct((NN, 1), jnp.float32),
        ],
    )(num0, brow, Wg, as_col, ad_col)


def _head_body(num_ref, b_ref, wc_ref, bc_ref, wkf_ref, sel_ref, bk_ref, out_ref):
    g = jax.nn.gelu(num_ref[...] + b_ref[...])
    cmat = (jnp.dot(g, wc_ref[...], preferred_element_type=jnp.float32)
            + bc_ref[...])
    weighted = cmat * wkf_ref[...]
    out_ref[...] = jnp.dot(sel_ref[...], weighted,
                           preferred_element_type=jnp.float32) + bk_ref[0, 0]


def _head(num1, brow, Wc, bcrow, wkf, sel, bkr):
    full = lambda s: pl.BlockSpec(s, lambda: tuple(0 for _ in s))
    return pl.pallas_call(
        _head_body,
        in_specs=[
            full((NN, D)),
            full((1, D)),
            full((D, NC)),
            full((1, NC)),
            full((NN, NC)),
            full((B, NN)),
            full((1, 1)),
        ],
        out_specs=full((B, NC)),
        out_shape=jax.ShapeDtypeStruct((B, NC), jnp.float32),
    )(num1, brow, Wc, bcrow, wkf, sel, bkr)


_sage_sc_deg = _make_sage_sc(with_deg=True)
_sage_sc = _make_sage_sc(with_deg=False)
_gat_sc = _make_gat_sc()


def kernel(x, roi_edge_index, net_edge_index, mask, W_root0, W_neigh0, b0,
           W_root1, W_neigh1, b1, pool_logits, Wg0, att_s0, att_d0, bg0,
           Wg1, att_s1, att_d1, bg1, Wc, bc, Wk, bk):
    f32 = jnp.float32

    src = roi_edge_index[0].astype(jnp.int32)
    dst = roi_edge_index[1].astype(jnp.int32)
    pad = E_PAD - E_ROI
    src_p = jnp.concatenate([src, jnp.zeros((pad,), jnp.int32)])
    dst_p = jnp.concatenate([dst, jnp.full((pad,), N_TOT, jnp.int32)])
    pk = (src_p << 15) | dst_p
    pkf = lax.bitcast_convert_type(pk, jnp.float32)

    srcn = net_edge_index[0].astype(jnp.int32)
    dstn = net_edge_index[1].astype(jnp.int32)

    z2 = jnp.zeros((ZROWS, QW), f32)
    z1 = jnp.zeros((ZROWS,), f32)
    z2g = jnp.zeros((NPT, HALF), f32)
    z1g = jnp.zeros((NPT,), f32)

    b0r = b0.reshape(1, H)
    b1r = b1.reshape(1, D)
    bg0r = bg0.reshape(1, H)
    bg1r = bg1.reshape(1, D)
    bcr = bc.reshape(1, NC)
    as0c = att_s0.reshape(H, 1)
    ad0c = att_d0.reshape(H, 1)
    as1c = att_s1.reshape(D, 1)
    ad1c = att_d1.reshape(D, 1)
    wkf = jnp.tile(Wk[:, 0], B).reshape(NN, 1) * jnp.ones((1, NC), f32)
    sel = jnp.repeat(jnp.eye(B, dtype=f32), N_NET, axis=1)

    agg0, deg_pad = _sage_sc_deg(x.reshape(NQ * N_TOT, QW), pkf, z2, z1)
    degc = deg_pad[:N_TOT].reshape(N_TOT, 1)
    h1 = _sage_dense(x, agg0, degc, W_root0, W_neigh0.reshape(NQ, QW, H), b0r)
    agg1 = _sage_sc(h1.reshape(NQ * N_TOT, QW), pkf, z2, z1)

    hp0, as0, ad0 = _sage_dense_pool(h1, agg1, degc, W_root1,
                                     W_neigh1.reshape(NQ, QW, H), b1r,
                                     mask, pool_logits, Wg0, as0c, ad0c)

    num0 = _gat_sc(hp0.reshape(NUM_CORES * NN, HALF), srcn, dstn,
                   as0.reshape(NN), ad0.reshape(NN), z2g, z1g)
    hp1, as1, ad1 = _gat_mid(num0, bg0r, Wg1, as1c, ad1c)
    num1 = _gat_sc(hp1.reshape(NUM_CORES * NN, HALF), srcn, dstn,
                   as1.reshape(NN), ad1.reshape(NN), z2g, z1g)

    return _head(num1, bg1r, Wc, bcr, wkf, sel, bk.reshape(1, 1))


# speedup vs baseline: 1.1197x; 1.1197x over previous
import jax
import jax.numpy as jnp
from jax import lax
from jax.experimental import pallas as pl
from jax.experimental.pallas import tpu as pltpu
from jax.experimental.pallas import tpu_sc as plsc

B = 64
N_ROI = 347
N_NET = 16
D = 128
H = 128
HALF = 64
NC = 4
N_TOT = B * N_ROI
NN = B * N_NET
E_ROI = N_TOT * 16
E_NET = NN * 16

NUM_CORES = 2
NUM_SUBCORES = 16

CH = 1024
NCHUNK = 22
EPT = CH * NCHUNK
E_PAD = EPT * NUM_SUBCORES
N_ACC = 22528
ZROWS = N_ACC // NUM_SUBCORES
QW = 32
NQ = 4

ENT = E_NET // NUM_SUBCORES
NPT = NN // NUM_SUBCORES


def _sc_mesh():
    return plsc.VectorSubcoreMesh(
        core_axis_name="c", subcore_axis_name="s",
        num_cores=NUM_CORES, num_subcores=NUM_SUBCORES)


_SC_PARAMS = pltpu.CompilerParams(use_tc_tiling_on_sc=False,
                                  needs_layout_passes=False)


def _make_sage_sc(with_deg: bool):
    out_type = [jax.ShapeDtypeStruct((N_ACC, D), jnp.float32)]
    if with_deg:
        out_type.append(jax.ShapeDtypeStruct((N_ACC,), jnp.float32))
    scratch = [
        [pltpu.VMEM((CH,), jnp.float32) for _ in range(2)],
        [pltpu.VMEM((CH,), jnp.int32) for _ in range(2)],
        [pltpu.VMEM((CH,), jnp.int32) for _ in range(2)],
        [pltpu.VMEM((CH, QW), jnp.float32) for _ in range(2)],
        pltpu.VMEM_SHARED((N_ACC, QW), jnp.float32),
        [pltpu.SemaphoreType.DMA for _ in range(2)],
    ]
    if with_deg:
        scratch += [
            pltpu.VMEM((CH,), jnp.float32),
            pltpu.VMEM_SHARED((N_ACC,), jnp.float32),
        ]

    def body(xs_hbm, pk_hbm, z2_hbm, z1_hbm, out_hbm, *rest):
        if with_deg:
            deg_hbm, pkv, srcv, dstv, rows, acc, sem, onesv, dacc = rest
        else:
            pkv, srcv, dstv, rows, acc, sem = rest
        c = lax.axis_index("c")
        s = lax.axis_index("s")
        base = s * EPT
        z0 = pl.multiple_of(s * ZROWS, ZROWS)

        if with_deg:
            @pl.when(c == 0)
            def _():
                @pl.loop(0, CH // 16)
                def _(i):
                    onesv[pl.ds(i * 16, 16)] = jnp.full((16,), 1.0, jnp.float32)

        for q in range(2):
            qidx = 2 * c + q
            pltpu.sync_copy(z2_hbm, acc.at[pl.ds(z0, ZROWS)])
            if with_deg and q == 0:
                @pl.when(c == 0)
                def _():
                    pltpu.sync_copy(z1_hbm, dacc.at[pl.ds(z0, ZROWS)])
            plsc.subcore_barrier()

            qoff = jnp.full((16,), 0, jnp.int32) + qidx * N_TOT

            def load_unpack_start(kk, b):
                off = pl.multiple_of(base + kk * CH, CH)
                pltpu.sync_copy(pk_hbm.at[pl.ds(off, CH)], pkv[b])

                @pl.loop(0, CH // 16, unroll=4)
                def _(g):
                    sl = pl.ds(g * 16, 16)
                    p = plsc.bitcast(pkv[b][sl], jnp.int32)
                    srcv[b][sl] = (p >> 15) + qoff
                    dstv[b][sl] = p & 32767

                return pltpu.async_copy(xs_hbm.at[srcv[b]], rows[b], sem[b])

            def wait_scatter(b):
                pltpu.make_async_copy(xs_hbm.at[srcv[b]], rows[b], sem[b]).wait()
                pltpu.sync_copy(rows[b], acc.at[dstv[b]], add=True)
                if with_deg and q == 0:
                    @pl.when(c == 0)
                    def _():
                        pltpu.sync_copy(onesv, dacc.at[dstv[b]], add=True)

            load_unpack_start(0, 0)

            @pl.loop(0, NCHUNK // 2)
            def _(i):
                load_unpack_start(2 * i + 1, 1)
                wait_scatter(0)

                @pl.when(i < NCHUNK // 2 - 1)
                def _():
                    load_unpack_start(2 * i + 2, 0)
                wait_scatter(1)

            plsc.subcore_barrier()
            pltpu.sync_copy(acc.at[pl.ds(z0, ZROWS)],
                            out_hbm.at[pl.ds(z0, ZROWS), pl.ds(qidx * QW, QW)])
            if with_deg and q == 0:
                @pl.when(c == 0)
                def _():
                    pltpu.sync_copy(dacc.at[pl.ds(z0, ZROWS)], deg_hbm.at[pl.ds(z0, ZROWS)])

    return pl.kernel(body, out_type=tuple(out_type) if with_deg else out_type[0],
                     mesh=_sc_mesh(), scratch_types=scratch,
                     compiler_params=_SC_PARAMS)


def _make_gat_sc():
    out_type = jax.ShapeDtypeStruct((NN, D), jnp.float32)
    scratch = [
        pltpu.VMEM((ENT,), jnp.int32),
        pltpu.VMEM((ENT,), jnp.int32),
        pltpu.VMEM((ENT,), jnp.int32),
        pltpu.VMEM((NN,), jnp.float32),
        pltpu.VMEM((NN,), jnp.float32),
        pltpu.VMEM((ENT,), jnp.float32),
        pltpu.VMEM((ENT,), jnp.float32),
        pltpu.VMEM((NN,), jnp.float32),
        pltpu.VMEM((ENT, HALF), jnp.float32),
        pltpu.VMEM_SHARED((NN,), jnp.float32),
        pltpu.VMEM_SHARED((NN, HALF), jnp.float32),
        pltpu.SemaphoreType.DMA,
    ]

    def body(hs_hbm, src_hbm, dst_hbm, asv_hbm, adv_hbm,
             z2_hbm, z1_hbm, out_hbm,
             srcv, gsrcv, dstv, asv, adv, exbuf, coefbuf, denv, rows,
             dacc, nacc, sem):
        c = lax.axis_index("c")
        s = lax.axis_index("s")
        base = s * ENT

        pltpu.sync_copy(src_hbm.at[pl.ds(base, ENT)], srcv)
        pltpu.sync_copy(dst_hbm.at[pl.ds(base, ENT)], dstv)
        pltpu.sync_copy(asv_hbm, asv)
        pltpu.sync_copy(adv_hbm, adv)

        coff = jnp.full((16,), 0, jnp.int32) + c

        @pl.loop(0, ENT // 16, unroll=4)
        def _(g):
            sl = pl.ds(g * 16, 16)
            gsrcv[sl] = (srcv[sl] << 1) + coff

        gat = pltpu.async_copy(hs_hbm.at[gsrcv], rows, sem)

        pltpu.sync_copy(z1_hbm, dacc.at[pl.ds(s * NPT, NPT)])
        pltpu.sync_copy(z2_hbm, nacc.at[pl.ds(s * NPT, NPT)])
        plsc.subcore_barrier()

        @pl.loop(0, ENT // 16)
        def _(g):
            sl = pl.ds(g * 16, 16)
            e = plsc.load_gather(asv, [srcv[sl]]) + plsc.load_gather(adv, [dstv[sl]])
            e = jnp.where(e >= 0, e, 0.2 * e)
            exbuf[sl] = jnp.exp(e)

        pltpu.sync_copy(exbuf, dacc.at[dstv], add=True)
        plsc.subcore_barrier()
        pltpu.sync_copy(dacc, denv)

        @pl.loop(0, ENT // 16)
        def _(g):
            sl = pl.ds(g * 16, 16)
            den = plsc.load_gather(denv, [dstv[sl]])
            coefbuf[sl] = exbuf[sl] / (den + 1e-16)

        gat.wait()

        @pl.loop(0, ENT)
        def _(j):
            cf = plsc.load_gather(coefbuf, [jnp.full((16,), 0, jnp.int32) + j])
            for jj in range(HALF // 16):
                sl2 = pl.ds(jj * 16, 16)
                rows[j, sl2] = rows[j, sl2] * cf

        pltpu.sync_copy(rows, nacc.at[dstv], add=True)
        plsc.subcore_barrier()
        n0 = pl.multiple_of(s * NPT, NPT)
        pltpu.sync_copy(nacc.at[pl.ds(n0, NPT)],
                        out_hbm.at[pl.ds(n0, NPT), pl.ds(c * HALF, HALF)])

    return pl.kernel(body, out_type=out_type, mesh=_sc_mesh(),
                     scratch_types=scratch, compiler_params=_SC_PARAMS)


BN = 2776
BPB = BN // N_ROI


def _sage_dense_body(x_ref, agg_ref, deg_ref, wr_ref, wn_ref, b_ref,
                     out_ref, outq_ref):
    inv = 1.0 / jnp.maximum(deg_ref[...], 1.0)
    h = (jnp.dot(x_ref[...], wr_ref[...], preferred_element_type=jnp.float32)
         + jnp.dot(agg_ref[...] * inv, wn_ref[...],
                   preferred_element_type=jnp.float32)
         + b_ref[...])
    h = jax.nn.gelu(h)
    out_ref[...] = h
    for q in range(NQ):
        outq_ref[q] = h[:, QW * q:QW * (q + 1)]


def _sage_dense(xin, agg, degc, Wr, Wn, brow):
    return pl.pallas_call(
        _sage_dense_body,
        grid=(N_TOT // BN,),
        in_specs=[
            pl.BlockSpec((BN, D), lambda i: (i, 0)),
            pl.BlockSpec((BN, D), lambda i: (i, 0)),
            pl.BlockSpec((BN, 1), lambda i: (i, 0)),
            pl.BlockSpec((D, H), lambda i: (0, 0)),
            pl.BlockSpec((D, H), lambda i: (0, 0)),
            pl.BlockSpec((1, H), lambda i: (0, 0)),
        ],
        out_specs=[
            pl.BlockSpec((BN, D), lambda i: (i, 0)),
            pl.BlockSpec((NQ, BN, QW), lambda i: (0, i, 0)),
        ],
        out_shape=[
            jax.ShapeDtypeStruct((N_TOT, D), jnp.float32),
            jax.ShapeDtypeStruct((NQ, N_TOT, QW), jnp.float32),
        ],
    )(xin, agg, degc, Wr, Wn, brow)


def _sage_pool_body(x_ref, agg_ref, deg_ref, wr_ref, wn_ref, b_ref,
                    mask_ref, logit_ref, wg_ref, as_ref, ad_ref,
                    hp_ref, asv_ref, adv_ref, alpha_ref):
    @pl.when(pl.program_id(0) == 0)
    def _():
        logits = jnp.where(mask_ref[...] == 0, -20.0, logit_ref[...])
        m = jnp.max(logits, axis=0, keepdims=True)
        ex = jnp.exp(logits - m)
        alpha_ref[...] = ex / jnp.sum(ex, axis=0, keepdims=True)

    inv = 1.0 / jnp.maximum(deg_ref[...], 1.0)
    h = (jnp.dot(x_ref[...], wr_ref[...], preferred_element_type=jnp.float32)
         + jnp.dot(agg_ref[...] * inv, wn_ref[...],
                   preferred_element_type=jnp.float32)
         + b_ref[...])
    h = jax.nn.gelu(h)
    alpha = alpha_ref[...]
    zs = [lax.dot_general(alpha, h[N_ROI * b:N_ROI * (b + 1), :],
                          (((0,), (0,)), ((), ())),
                          preferred_element_type=jnp.float32)
          for b in range(BPB)]
    z = jnp.concatenate(zs, axis=0)
    hp = jnp.dot(z, wg_ref[...], preferred_element_type=jnp.float32)
    asv_ref[...] = jnp.dot(hp, as_ref[...], preferred_element_type=jnp.float32)
    adv_ref[...] = jnp.dot(hp, ad_ref[...], preferred_element_type=jnp.float32)
    hp_ref[...] = hp


def _sage_dense_pool(xin, agg, degc, Wr, Wn, brow, mask, pool_logits,
                     Wg0, as_col, ad_col):
    zrows = BPB * N_NET
    return pl.pallas_call(
        _sage_pool_body,
        grid=(N_TOT // BN,),
        in_specs=[
            pl.BlockSpec((BN, D), lambda i: (i, 0)),
            pl.BlockSpec((BN, D), lambda i: (i, 0)),
            pl.BlockSpec((BN, 1), lambda i: (i, 0)),
            pl.BlockSpec((D, H), lambda i: (0, 0)),
            pl.BlockSpec((D, H), lambda i: (0, 0)),
            pl.BlockSpec((1, H), lambda i: (0, 0)),
            pl.BlockSpec((N_ROI, N_NET), lambda i: (0, 0)),
            pl.BlockSpec((N_ROI, N_NET), lambda i: (0, 0)),
            pl.BlockSpec((D, H), lambda i: (0, 0)),
            pl.BlockSpec((H, 1), lambda i: (0, 0)),
            pl.BlockSpec((H, 1), lambda i: (0, 0)),
        ],
        out_specs=[
            pl.BlockSpec((zrows, D), lambda i: (i, 0)),
            pl.BlockSpec((zrows, 1), lambda i: (i, 0)),
            pl.BlockSpec((zrows, 1), lambda i: (i, 0)),
        ],
        out_shape=[
            jax.ShapeDtypeStruct((NN, D), jnp.float32),
            jax.ShapeDtypeStruct((NN, 1), jnp.float32),
            jax.ShapeDtypeStruct((NN, 1), jnp.float32),
        ],
        scratch_shapes=[pltpu.VMEM((N_ROI, N_NET), jnp.float32)],
    )(xin, agg, degc, Wr, Wn, brow, mask, pool_logits, Wg0, as_col, ad_col)


def _gat_mid_body(num_ref, b_ref, wg_ref, as_ref, ad_ref,
                  hp_ref, asv_ref, adv_ref):
    g = jax.nn.gelu(num_ref[...] + b_ref[...])
    h = jnp.dot(g, wg_ref[...], preferred_element_type=jnp.float32)
    asv_ref[...] = jnp.dot(h, as_ref[...], preferred_element_type=jnp.float32)
    adv_ref[...] = jnp.dot(h, ad_ref[...], preferred_element_type=jnp.float32)
    hp_ref[...] = h


def _gat_mid(num0, brow, Wg, as_col, ad_col):
    full = lambda s: pl.BlockSpec(s, lambda: tuple(0 for _ in s))
    return pl.pallas_call(
        _gat_mid_body,
        in_specs=[
            full((NN, D)),
            full((1, H)),
            full((D, H)),
            full((D, 1)),
            full((D, 1)),
        ],
        out_specs=[
            full((NN, D)),
            full((NN, 1)),
            full((NN, 1)),
        ],
        out_shape=[
            jax.ShapeDtypeStruct((NN, D), jnp.float32),
            jax.ShapeDtypeStruct((NN, 1), jnp.float32),
            jax.ShapeDtypeStruct((NN, 1), jnp.float32),
        ],
    )(num0, brow, Wg, as_col, ad_col)


def _head_body(num_ref, b_ref, wc_ref, bc_ref, wkf_ref, sel_ref, bk_ref, out_ref):
    g = jax.nn.gelu(num_ref[...] + b_ref[...])
    cmat = (jnp.dot(g, wc_ref[...], preferred_element_type=jnp.float32)
            + bc_ref[...])
    weighted = cmat * wkf_ref[...]
    out_ref[...] = jnp.dot(sel_ref[...], weighted,
                           preferred_element_type=jnp.float32) + bk_ref[0, 0]


def _head(num1, brow, Wc, bcrow, wkf, sel, bkr):
    full = lambda s: pl.BlockSpec(s, lambda: tuple(0 for _ in s))
    return pl.pallas_call(
        _head_body,
        in_specs=[
            full((NN, D)),
            full((1, D)),
            full((D, NC)),
            full((1, NC)),
            full((NN, NC)),
            full((B, NN)),
            full((1, 1)),
        ],
        out_specs=full((B, NC)),
        out_shape=jax.ShapeDtypeStruct((B, NC), jnp.float32),
    )(num1, brow, Wc, bcrow, wkf, sel, bkr)


_sage_sc_deg = _make_sage_sc(with_deg=True)
_sage_sc = _make_sage_sc(with_deg=False)
_gat_sc = _make_gat_sc()


def kernel(x, roi_edge_index, net_edge_index, mask, W_root0, W_neigh0, b0,
           W_root1, W_neigh1, b1, pool_logits, Wg0, att_s0, att_d0, bg0,
           Wg1, att_s1, att_d1, bg1, Wc, bc, Wk, bk):
    f32 = jnp.float32

    src = roi_edge_index[0].astype(jnp.int32)
    dst = roi_edge_index[1].astype(jnp.int32)
    pad = E_PAD - E_ROI
    src_p = jnp.concatenate([src, jnp.zeros((pad,), jnp.int32)])
    dst_p = jnp.concatenate([dst, jnp.full((pad,), N_TOT, jnp.int32)])
    pk = (src_p << 15) | dst_p
    pkf = lax.bitcast_convert_type(pk, jnp.float32)

    srcn = net_edge_index[0].astype(jnp.int32)
    dstn = net_edge_index[1].astype(jnp.int32)

    z2 = jnp.zeros((ZROWS, QW), f32)
    z1 = jnp.zeros((ZROWS,), f32)
    z2g = jnp.zeros((NPT, HALF), f32)
    z1g = jnp.zeros((NPT,), f32)

    b0r = b0.reshape(1, H)
    b1r = b1.reshape(1, D)
    bg0r = bg0.reshape(1, H)
    bg1r = bg1.reshape(1, D)
    bcr = bc.reshape(1, NC)
    as0c = att_s0.reshape(H, 1)
    ad0c = att_d0.reshape(H, 1)
    as1c = att_s1.reshape(D, 1)
    ad1c = att_d1.reshape(D, 1)
    wkf = jnp.tile(Wk[:, 0], B).reshape(NN, 1) * jnp.ones((1, NC), f32)
    sel = jnp.repeat(jnp.eye(B, dtype=f32), N_NET, axis=1)

    xs0 = x.reshape(N_TOT, NQ, QW).transpose(1, 0, 2).reshape(NQ * N_TOT, QW)
    agg0, deg_pad = _sage_sc_deg(xs0, pkf, z2, z1)
    degc = deg_pad[:N_TOT].reshape(N_TOT, 1)
    h1, h1q = _sage_dense(x, agg0, degc, W_root0, W_neigh0, b0r)
    agg1 = _sage_sc(h1q.reshape(NQ * N_TOT, QW), pkf, z2, z1)

    hp0, as0, ad0 = _sage_dense_pool(h1, agg1, degc, W_root1,
                                     W_neigh1, b1r, mask, pool_logits,
                                     Wg0, as0c, ad0c)

    num0 = _gat_sc(hp0.reshape(NUM_CORES * NN, HALF), srcn, dstn,
                   as0.reshape(NN), ad0.reshape(NN), z2g, z1g)
    hp1, as1, ad1 = _gat_mid(num0, bg0r, Wg1, as1c, ad1c)
    num1 = _gat_sc(hp1.reshape(NUM_CORES * NN, HALF), srcn, dstn,
                   as1.reshape(NN), ad1.reshape(NN), z2g, z1g)

    return _head(num1, bg1r, Wc, bcr, wkf, sel, bk.reshape(1, 1))


# speedup vs baseline: 1.1229x; 1.0029x over previous
import jax
import jax.numpy as jnp
from jax import lax
from jax.experimental import pallas as pl
from jax.experimental.pallas import tpu as pltpu
from jax.experimental.pallas import tpu_sc as plsc

B = 64
N_ROI = 347
N_NET = 16
D = 128
H = 128
HALF = 64
NC = 4
N_TOT = B * N_ROI
NN = B * N_NET
E_ROI = N_TOT * 16
E_NET = NN * 16

NUM_CORES = 2
NUM_SUBCORES = 16

CH = 1024
NCHUNK = 22
EPT = CH * NCHUNK
E_PAD = EPT * NUM_SUBCORES
N_ACC = 22528
ZROWS = N_ACC // NUM_SUBCORES
QW = 32
NQ = 4

ENT = E_NET // NUM_SUBCORES
NPT = NN // NUM_SUBCORES


def _sc_mesh():
    return plsc.VectorSubcoreMesh(
        core_axis_name="c", subcore_axis_name="s",
        num_cores=NUM_CORES, num_subcores=NUM_SUBCORES)


_SC_PARAMS = pltpu.CompilerParams(use_tc_tiling_on_sc=False,
                                  needs_layout_passes=False)


def _make_sage_sc(with_deg: bool):
    out_type = [jax.ShapeDtypeStruct((N_ACC, D), jnp.float32)]
    if with_deg:
        out_type.append(jax.ShapeDtypeStruct((N_ACC,), jnp.float32))
    scratch = [
        [pltpu.VMEM((CH,), jnp.float32) for _ in range(2)],
        [pltpu.VMEM((CH,), jnp.int32) for _ in range(2)],
        [pltpu.VMEM((CH,), jnp.int32) for _ in range(2)],
        [pltpu.VMEM((CH, QW), jnp.float32) for _ in range(2)],
        pltpu.VMEM_SHARED((N_ACC, QW), jnp.float32),
        [pltpu.SemaphoreType.DMA for _ in range(2)],
        pltpu.VMEM((ZROWS // 4, QW), jnp.float32),
    ]
    if with_deg:
        scratch += [
            pltpu.VMEM((CH,), jnp.float32),
            pltpu.VMEM((ZROWS // 4,), jnp.float32),
            pltpu.VMEM_SHARED((N_ACC,), jnp.float32),
        ]

    def body(xs_hbm, pk_hbm, out_hbm, *rest):
        if with_deg:
            deg_hbm, pkv, srcv, dstv, rows, acc, sem, zbuf, onesv, zv, dacc = rest
        else:
            pkv, srcv, dstv, rows, acc, sem, zbuf = rest
        c = lax.axis_index("c")
        s = lax.axis_index("s")
        base = s * EPT
        z0 = pl.multiple_of(s * ZROWS, ZROWS)
        ZB = ZROWS // 4

        @pl.loop(0, ZB * (QW // 16))
        def _(i):
            zbuf[i // (QW // 16), pl.ds((i % (QW // 16)) * 16, 16)] = (
                jnp.zeros((16,), jnp.float32))

        if with_deg:
            @pl.when(c == 0)
            def _():
                @pl.loop(0, CH // 16)
                def _(i):
                    onesv[pl.ds(i * 16, 16)] = jnp.full((16,), 1.0, jnp.float32)

                @pl.loop(0, ZB // 16)
                def _(i):
                    zv[pl.ds(i * 16, 16)] = jnp.zeros((16,), jnp.float32)

        for q in range(2):
            qidx = 2 * c + q
            for r in range(4):
                pltpu.sync_copy(zbuf, acc.at[pl.ds(z0 + r * ZB, ZB)])
            if with_deg and q == 0:
                @pl.when(c == 0)
                def _():
                    for r in range(4):
                        pltpu.sync_copy(zv, dacc.at[pl.ds(z0 + r * ZB, ZB)])
            plsc.subcore_barrier()

            qoff = jnp.full((16,), 0, jnp.int32) + qidx * N_TOT

            def load_unpack_start(kk, b):
                off = pl.multiple_of(base + kk * CH, CH)
                pltpu.sync_copy(pk_hbm.at[pl.ds(off, CH)], pkv[b])

                @pl.loop(0, CH // 16, unroll=4)
                def _(g):
                    sl = pl.ds(g * 16, 16)
                    p = plsc.bitcast(pkv[b][sl], jnp.int32)
                    srcv[b][sl] = (p >> 15) + qoff
                    dstv[b][sl] = p & 32767

                return pltpu.async_copy(xs_hbm.at[srcv[b]], rows[b], sem[b])

            def wait_scatter(b):
                pltpu.make_async_copy(xs_hbm.at[srcv[b]], rows[b], sem[b]).wait()
                pltpu.sync_copy(rows[b], acc.at[dstv[b]], add=True)
                if with_deg and q == 0:
                    @pl.when(c == 0)
                    def _():
                        pltpu.sync_copy(onesv, dacc.at[dstv[b]], add=True)

            load_unpack_start(0, 0)

            @pl.loop(0, NCHUNK // 2)
            def _(i):
                load_unpack_start(2 * i + 1, 1)
                wait_scatter(0)

                @pl.when(i < NCHUNK // 2 - 1)
                def _():
                    load_unpack_start(2 * i + 2, 0)
                wait_scatter(1)

            plsc.subcore_barrier()
            pltpu.sync_copy(acc.at[pl.ds(z0, ZROWS)],
                            out_hbm.at[pl.ds(z0, ZROWS), pl.ds(qidx * QW, QW)])
            if with_deg and q == 0:
                @pl.when(c == 0)
                def _():
                    pltpu.sync_copy(dacc.at[pl.ds(z0, ZROWS)], deg_hbm.at[pl.ds(z0, ZROWS)])

    return pl.kernel(body, out_type=tuple(out_type) if with_deg else out_type[0],
                     mesh=_sc_mesh(), scratch_types=scratch,
                     compiler_params=_SC_PARAMS)


def _make_gat_sc():
    out_type = jax.ShapeDtypeStruct((NN, D), jnp.float32)
    scratch = [
        pltpu.VMEM((ENT,), jnp.int32),
        pltpu.VMEM((ENT,), jnp.int32),
        pltpu.VMEM((ENT,), jnp.int32),
        pltpu.VMEM((NN,), jnp.float32),
        pltpu.VMEM((NN,), jnp.float32),
        pltpu.VMEM((ENT,), jnp.float32),
        pltpu.VMEM((ENT,), jnp.float32),
        pltpu.VMEM((NN,), jnp.float32),
        pltpu.VMEM((ENT, HALF), jnp.float32),
        pltpu.VMEM((NPT, HALF), jnp.float32),
        pltpu.VMEM((NPT,), jnp.float32),
        pltpu.VMEM_SHARED((NN,), jnp.float32),
        pltpu.VMEM_SHARED((NN, HALF), jnp.float32),
        pltpu.SemaphoreType.DMA,
    ]

    def body(hs_hbm, src_hbm, dst_hbm, asv_hbm, adv_hbm, out_hbm,
             srcv, gsrcv, dstv, asv, adv, exbuf, coefbuf, denv, rows,
             zbuf, zv, dacc, nacc, sem):
        c = lax.axis_index("c")
        s = lax.axis_index("s")
        base = s * ENT

        pltpu.sync_copy(src_hbm.at[pl.ds(base, ENT)], srcv)
        pltpu.sync_copy(dst_hbm.at[pl.ds(base, ENT)], dstv)
        pltpu.sync_copy(asv_hbm, asv)
        pltpu.sync_copy(adv_hbm, adv)

        coff = jnp.full((16,), 0, jnp.int32) + c

        @pl.loop(0, ENT // 16, unroll=4)
        def _(g):
            sl = pl.ds(g * 16, 16)
            gsrcv[sl] = (srcv[sl] << 1) + coff

        gat = pltpu.async_copy(hs_hbm.at[gsrcv], rows, sem)

        @pl.loop(0, NPT * (HALF // 16))
        def _(i):
            zbuf[i // (HALF // 16), pl.ds((i % (HALF // 16)) * 16, 16)] = (
                jnp.zeros((16,), jnp.float32))

        @pl.loop(0, NPT // 16)
        def _(i):
            zv[pl.ds(i * 16, 16)] = jnp.zeros((16,), jnp.float32)

        pltpu.sync_copy(zv, dacc.at[pl.ds(s * NPT, NPT)])
        pltpu.sync_copy(zbuf, nacc.at[pl.ds(s * NPT, NPT)])
        plsc.subcore_barrier()

        @pl.loop(0, ENT // 16)
        def _(g):
            sl = pl.ds(g * 16, 16)
            e = plsc.load_gather(asv, [srcv[sl]]) + plsc.load_gather(adv, [dstv[sl]])
            e = jnp.where(e >= 0, e, 0.2 * e)
            exbuf[sl] = jnp.exp(e)

        pltpu.sync_copy(exbuf, dacc.at[dstv], add=True)
        plsc.subcore_barrier()
        pltpu.sync_copy(dacc, denv)

        @pl.loop(0, ENT // 16)
        def _(g):
            sl = pl.ds(g * 16, 16)
            den = plsc.load_gather(denv, [dstv[sl]])
            coefbuf[sl] = exbuf[sl] / (den + 1e-16)

        gat.wait()

        @pl.loop(0, ENT)
        def _(j):
            cf = plsc.load_gather(coefbuf, [jnp.full((16,), 0, jnp.int32) + j])
            for jj in range(HALF // 16):
                sl2 = pl.ds(jj * 16, 16)
                rows[j, sl2] = rows[j, sl2] * cf

        pltpu.sync_copy(rows, nacc.at[dstv], add=True)
        plsc.subcore_barrier()
        n0 = pl.multiple_of(s * NPT, NPT)
        pltpu.sync_copy(nacc.at[pl.ds(n0, NPT)],
                        out_hbm.at[pl.ds(n0, NPT), pl.ds(c * HALF, HALF)])

    return pl.kernel(body, out_type=out_type, mesh=_sc_mesh(),
                     scratch_types=scratch, compiler_params=_SC_PARAMS)


BN = 2776
BPB = BN // N_ROI


def _sage_dense_body(x_ref, agg_ref, deg_ref, wr_ref, wn_ref, b_ref,
                     out_ref, outq_ref):
    inv = 1.0 / jnp.maximum(deg_ref[...], 1.0)
    h = (jnp.dot(x_ref[...], wr_ref[...], preferred_element_type=jnp.float32)
         + jnp.dot(agg_ref[...] * inv, wn_ref[...],
                   preferred_element_type=jnp.float32)
         + b_ref[...])
    h = jax.nn.gelu(h)
    out_ref[...] = h
    for q in range(NQ):
        outq_ref[q] = h[:, QW * q:QW * (q + 1)]


def _sage_dense(xin, agg, degc, Wr, Wn, brow):
    return pl.pallas_call(
        _sage_dense_body,
        grid=(N_TOT // BN,),
        in_specs=[
            pl.BlockSpec((BN, D), lambda i: (i, 0)),
            pl.BlockSpec((BN, D), lambda i: (i, 0)),
            pl.BlockSpec((BN, 1), lambda i: (i, 0)),
            pl.BlockSpec((D, H), lambda i: (0, 0)),
            pl.BlockSpec((D, H), lambda i: (0, 0)),
            pl.BlockSpec((1, H), lambda i: (0, 0)),
        ],
        out_specs=[
            pl.BlockSpec((BN, D), lambda i: (i, 0)),
            pl.BlockSpec((NQ, BN, QW), lambda i: (0, i, 0)),
        ],
        out_shape=[
            jax.ShapeDtypeStruct((N_TOT, D), jnp.float32),
            jax.ShapeDtypeStruct((NQ, N_TOT, QW), jnp.float32),
        ],
    )(xin, agg, degc, Wr, Wn, brow)


def _sage_pool_body(x_ref, agg_ref, deg_ref, wr_ref, wn_ref, b_ref,
                    mask_ref, logit_ref, wg_ref, as_ref, ad_ref,
                    hp_ref, asv_ref, adv_ref, alpha_ref):
    @pl.when(pl.program_id(0) == 0)
    def _():
        logits = jnp.where(mask_ref[...] == 0, -20.0, logit_ref[...])
        m = jnp.max(logits, axis=0, keepdims=True)
        ex = jnp.exp(logits - m)
        alpha_ref[...] = ex / jnp.sum(ex, axis=0, keepdims=True)

    inv = 1.0 / jnp.maximum(deg_ref[...], 1.0)
    h = (jnp.dot(x_ref[...], wr_ref[...], preferred_element_type=jnp.float32)
         + jnp.dot(agg_ref[...] * inv, wn_ref[...],
                   preferred_element_type=jnp.float32)
         + b_ref[...])
    h = jax.nn.gelu(h)
    alpha = alpha_ref[...]
    zs = [lax.dot_general(alpha, h[N_ROI * b:N_ROI * (b + 1), :],
                          (((0,), (0,)), ((), ())),
                          preferred_element_type=jnp.float32)
          for b in range(BPB)]
    z = jnp.concatenate(zs, axis=0)
    hp = jnp.dot(z, wg_ref[...], preferred_element_type=jnp.float32)
    asv_ref[...] = jnp.dot(hp, as_ref[...], preferred_element_type=jnp.float32)
    adv_ref[...] = jnp.dot(hp, ad_ref[...], preferred_element_type=jnp.float32)
    hp_ref[...] = hp


def _sage_dense_pool(xin, agg, degc, Wr, Wn, brow, mask, pool_logits,
                     Wg0, as_col, ad_col):
    zrows = BPB * N_NET
    return pl.pallas_call(
        _sage_pool_body,
        grid=(N_TOT // BN,),
        in_specs=[
            pl.BlockSpec((BN, D), lambda i: (i, 0)),
            pl.BlockSpec((BN, D), lambda i: (i, 0)),
            pl.BlockSpec((BN, 1), lambda i: (i, 0)),
            pl.BlockSpec((D, H), lambda i: (0, 0)),
            pl.BlockSpec((D, H), lambda i: (0, 0)),
            pl.BlockSpec((1, H), lambda i: (0, 0)),
            pl.BlockSpec((N_ROI, N_NET), lambda i: (0, 0)),
            pl.BlockSpec((N_ROI, N_NET), lambda i: (0, 0)),
            pl.BlockSpec((D, H), lambda i: (0, 0)),
            pl.BlockSpec((H, 1), lambda i: (0, 0)),
            pl.BlockSpec((H, 1), lambda i: (0, 0)),
        ],
        out_specs=[
            pl.BlockSpec((zrows, D), lambda i: (i, 0)),
            pl.BlockSpec((zrows, 1), lambda i: (i, 0)),
            pl.BlockSpec((zrows, 1), lambda i: (i, 0)),
        ],
        out_shape=[
            jax.ShapeDtypeStruct((NN, D), jnp.float32),
            jax.ShapeDtypeStruct((NN, 1), jnp.float32),
            jax.ShapeDtypeStruct((NN, 1), jnp.float32),
        ],
        scratch_shapes=[pltpu.VMEM((N_ROI, N_NET), jnp.float32)],
    )(xin, agg, degc, Wr, Wn, brow, mask, pool_logits, Wg0, as_col, ad_col)


def _gat_mid_body(num_ref, b_ref, wg_ref, as_ref, ad_ref,
                  hp_ref, asv_ref, adv_ref):
    g = jax.nn.gelu(num_ref[...] + b_ref[...])
    h = jnp.dot(g, wg_ref[...], preferred_element_type=jnp.float32)
    asv_ref[...] = jnp.dot(h, as_ref[...], preferred_element_type=jnp.float32)
    adv_ref[...] = jnp.dot(h, ad_ref[...], preferred_element_type=jnp.float32)
    hp_ref[...] = h


def _gat_mid(num0, brow, Wg, as_col, ad_col):
    full = lambda s: pl.BlockSpec(s, lambda: tuple(0 for _ in s))
    return pl.pallas_call(
        _gat_mid_body,
        in_specs=[
            full((NN, D)),
            full((1, H)),
            full((D, H)),
            full((D, 1)),
            full((D, 1)),
        ],
        out_specs=[
            full((NN, D)),
            full((NN, 1)),
            full((NN, 1)),
        ],
        out_shape=[
            jax.ShapeDtypeStruct((NN, D), jnp.float32),
            jax.ShapeDtypeStruct((NN, 1), jnp.float32),
            jax.ShapeDtypeStruct((NN, 1), jnp.float32),
        ],
    )(num0, brow, Wg, as_col, ad_col)


def _head_body(num_ref, b_ref, wc_ref, bc_ref, wkf_ref, sel_ref, bk_ref, out_ref):
    g = jax.nn.gelu(num_ref[...] + b_ref[...])
    cmat = (jnp.dot(g, wc_ref[...], preferred_element_type=jnp.float32)
            + bc_ref[...])
    weighted = cmat * wkf_ref[...]
    out_ref[...] = jnp.dot(sel_ref[...], weighted,
                           preferred_element_type=jnp.float32) + bk_ref[0, 0]


def _head(num1, brow, Wc, bcrow, wkf, sel, bkr):
    full = lambda s: pl.BlockSpec(s, lambda: tuple(0 for _ in s))
    return pl.pallas_call(
        _head_body,
        in_specs=[
            full((NN, D)),
            full((1, D)),
            full((D, NC)),
            full((1, NC)),
            full((NN, NC)),
            full((B, NN)),
            full((1, 1)),
        ],
        out_specs=full((B, NC)),
        out_shape=jax.ShapeDtypeStruct((B, NC), jnp.float32),
    )(num1, brow, Wc, bcrow, wkf, sel, bkr)


_sage_sc_deg = _make_sage_sc(with_deg=True)
_sage_sc = _make_sage_sc(with_deg=False)
_gat_sc = _make_gat_sc()


def kernel(x, roi_edge_index, net_edge_index, mask, W_root0, W_neigh0, b0,
           W_root1, W_neigh1, b1, pool_logits, Wg0, att_s0, att_d0, bg0,
           Wg1, att_s1, att_d1, bg1, Wc, bc, Wk, bk):
    f32 = jnp.float32

    src = roi_edge_index[0].astype(jnp.int32)
    dst = roi_edge_index[1].astype(jnp.int32)
    pad = E_PAD - E_ROI
    src_p = jnp.concatenate([src, jnp.zeros((pad,), jnp.int32)])
    dst_p = jnp.concatenate([dst, jnp.full((pad,), N_TOT, jnp.int32)])
    pk = (src_p << 15) | dst_p
    pkf = lax.bitcast_convert_type(pk, jnp.float32)

    srcn = net_edge_index[0].astype(jnp.int32)
    dstn = net_edge_index[1].astype(jnp.int32)

    b0r = b0.reshape(1, H)
    b1r = b1.reshape(1, D)
    bg0r = bg0.reshape(1, H)
    bg1r = bg1.reshape(1, D)
    bcr = bc.reshape(1, NC)
    as0c = att_s0.reshape(H, 1)
    ad0c = att_d0.reshape(H, 1)
    as1c = att_s1.reshape(D, 1)
    ad1c = att_d1.reshape(D, 1)
    wkf = jnp.tile(Wk[:, 0], B).reshape(NN, 1) * jnp.ones((1, NC), f32)
    sel = jnp.repeat(jnp.eye(B, dtype=f32), N_NET, axis=1)

    xs0 = x.reshape(N_TOT, NQ, QW).transpose(1, 0, 2).reshape(NQ * N_TOT, QW)
    agg0, deg_pad = _sage_sc_deg(xs0, pkf)
    degc = deg_pad[:N_TOT].reshape(N_TOT, 1)
    h1, h1q = _sage_dense(x, agg0, degc, W_root0, W_neigh0, b0r)
    agg1 = _sage_sc(h1q.reshape(NQ * N_TOT, QW), pkf)

    hp0, as0, ad0 = _sage_dense_pool(h1, agg1, degc, W_root1,
                                     W_neigh1, b1r, mask, pool_logits,
                                     Wg0, as0c, ad0c)

    num0 = _gat_sc(hp0.reshape(NUM_CORES * NN, HALF), srcn, dstn,
                   as0.reshape(NN), ad0.reshape(NN))
    hp1, as1, ad1 = _gat_mid(num0, bg0r, Wg1, as1c, ad1c)
    num1 = _gat_sc(hp1.reshape(NUM_CORES * NN, HALF), srcn, dstn,
                   as1.reshape(NN), ad1.reshape(NN))

    return _head(num1, bg1r, Wc, bcr, wkf, sel, bk.reshape(1, 1))


# speedup vs baseline: 1.1552x; 1.0288x over previous
import jax
import jax.numpy as jnp
from jax import lax
from jax.experimental import pallas as pl
from jax.experimental.pallas import tpu as pltpu
from jax.experimental.pallas import tpu_sc as plsc

B = 64
N_ROI = 347
N_NET = 16
D = 128
H = 128
HALF = 64
NC = 4
N_TOT = B * N_ROI
NN = B * N_NET
E_ROI = N_TOT * 16
E_NET = NN * 16

NUM_CORES = 2
NUM_SUBCORES = 16

CH = 1024
NCHUNK = 22
EPT = CH * NCHUNK
E_PAD = EPT * NUM_SUBCORES
N_ACC = 22528
ZROWS = N_ACC // NUM_SUBCORES
QW = 32
NQ = 4

ENT = E_NET // NUM_SUBCORES
NPT = NN // NUM_SUBCORES


def _sc_mesh():
    return plsc.VectorSubcoreMesh(
        core_axis_name="c", subcore_axis_name="s",
        num_cores=NUM_CORES, num_subcores=NUM_SUBCORES)


_SC_PARAMS = pltpu.CompilerParams(use_tc_tiling_on_sc=False,
                                  needs_layout_passes=False)


def _make_sage_sc(with_deg: bool):
    out_type = [jax.ShapeDtypeStruct((N_ACC, D), jnp.float32)]
    if with_deg:
        out_type.append(jax.ShapeDtypeStruct((N_ACC,), jnp.float32))
    scratch = [
        [pltpu.VMEM((CH,), jnp.float32) for _ in range(2)],
        [pltpu.VMEM((CH,), jnp.int32) for _ in range(2)],
        [pltpu.VMEM((CH,), jnp.int32) for _ in range(2)],
        [pltpu.VMEM((CH, QW), jnp.float32) for _ in range(2)],
        pltpu.VMEM_SHARED((N_ACC, QW), jnp.float32),
        [pltpu.SemaphoreType.DMA for _ in range(2)],
        [pltpu.SemaphoreType.DMA for _ in range(2)],
        pltpu.VMEM((ZROWS // 8, QW), jnp.float32),
    ]
    if with_deg:
        scratch += [
            pltpu.VMEM((CH,), jnp.float32),
            pltpu.VMEM((ZROWS // 8,), jnp.float32),
            pltpu.VMEM_SHARED((N_ACC,), jnp.float32),
        ]

    def body(xs_hbm, pk_hbm, out_hbm, *rest):
        if with_deg:
            deg_hbm, pkv, srcv, dstv, rows, acc, sem, psem, zbuf, onesv, zv, dacc = rest
        else:
            pkv, srcv, dstv, rows, acc, sem, psem, zbuf = rest
        c = lax.axis_index("c")
        s = lax.axis_index("s")
        base = pl.multiple_of(s * EPT, CH)
        z0 = pl.multiple_of(s * ZROWS, ZROWS)
        ZB = ZROWS // 8

        def pk_fetch(kk, b):
            off = pl.multiple_of(base + kk * CH, CH)
            return pltpu.async_copy(pk_hbm.at[pl.ds(off, CH)], pkv[b], psem[b])

        def pk_wait(kk, b):
            off = pl.multiple_of(base + kk * CH, CH)
            pltpu.make_async_copy(pk_hbm.at[pl.ds(off, CH)], pkv[b], psem[b]).wait()

        @pl.loop(0, ZB * (QW // 16))
        def _(i):
            zbuf[i // (QW // 16), pl.ds((i % (QW // 16)) * 16, 16)] = (
                jnp.zeros((16,), jnp.float32))

        if with_deg:
            @pl.when(c == 0)
            def _():
                @pl.loop(0, CH // 16)
                def _(i):
                    onesv[pl.ds(i * 16, 16)] = jnp.full((16,), 1.0, jnp.float32)

                @pl.loop(0, ZB // 16)
                def _(i):
                    zv[pl.ds(i * 16, 16)] = jnp.zeros((16,), jnp.float32)

        for q in range(2):
            qidx = 2 * c + q
            for r in range(8):
                pltpu.sync_copy(zbuf, acc.at[pl.ds(z0 + r * ZB, ZB)])
            if with_deg and q == 0:
                @pl.when(c == 0)
                def _():
                    for r in range(8):
                        pltpu.sync_copy(zv, dacc.at[pl.ds(z0 + r * ZB, ZB)])
            plsc.subcore_barrier()

            qoff = jnp.full((16,), 0, jnp.int32) + qidx * N_TOT

            def load_unpack_start(kk, b):
                pk_wait(kk, b)

                @pl.loop(0, CH // 16, unroll=4)
                def _(g):
                    sl = pl.ds(g * 16, 16)
                    p = plsc.bitcast(pkv[b][sl], jnp.int32)
                    srcv[b][sl] = (p >> 15) + qoff
                    dstv[b][sl] = p & 32767

                @pl.when(kk + 2 < NCHUNK)
                def _():
                    pk_fetch(kk + 2, b)

                return pltpu.async_copy(xs_hbm.at[srcv[b]], rows[b], sem[b])

            def wait_scatter(b):
                pltpu.make_async_copy(xs_hbm.at[srcv[b]], rows[b], sem[b]).wait()
                pltpu.sync_copy(rows[b], acc.at[dstv[b]], add=True)
                if with_deg and q == 0:
                    @pl.when(c == 0)
                    def _():
                        pltpu.sync_copy(onesv, dacc.at[dstv[b]], add=True)

            pk_fetch(0, 0)
            pk_fetch(1, 1)
            load_unpack_start(0, 0)

            @pl.loop(0, NCHUNK // 2)
            def _(i):
                load_unpack_start(2 * i + 1, 1)
                wait_scatter(0)

                @pl.when(i < NCHUNK // 2 - 1)
                def _():
                    load_unpack_start(2 * i + 2, 0)
                wait_scatter(1)

            plsc.subcore_barrier()
            pltpu.sync_copy(acc.at[pl.ds(z0, ZROWS)],
                            out_hbm.at[pl.ds(z0, ZROWS), pl.ds(qidx * QW, QW)])
            if with_deg and q == 0:
                @pl.when(c == 0)
                def _():
                    pltpu.sync_copy(dacc.at[pl.ds(z0, ZROWS)], deg_hbm.at[pl.ds(z0, ZROWS)])

    return pl.kernel(body, out_type=tuple(out_type) if with_deg else out_type[0],
                     mesh=_sc_mesh(), scratch_types=scratch,
                     compiler_params=_SC_PARAMS)


def _make_gat_sc():
    out_type = jax.ShapeDtypeStruct((NN, D), jnp.float32)
    scratch = [
        pltpu.VMEM((ENT,), jnp.int32),
        pltpu.VMEM((ENT,), jnp.int32),
        pltpu.VMEM((ENT,), jnp.int32),
        pltpu.VMEM((NN,), jnp.float32),
        pltpu.VMEM((NN,), jnp.float32),
        pltpu.VMEM((ENT,), jnp.float32),
        pltpu.VMEM((ENT,), jnp.float32),
        pltpu.VMEM((NN,), jnp.float32),
        pltpu.VMEM((ENT, HALF), jnp.float32),
        pltpu.VMEM((NPT, HALF), jnp.float32),
        pltpu.VMEM((NPT,), jnp.float32),
        pltpu.VMEM_SHARED((NN,), jnp.float32),
        pltpu.VMEM_SHARED((NN, HALF), jnp.float32),
        pltpu.SemaphoreType.DMA,
    ]

    def body(hs_hbm, src_hbm, dst_hbm, asv_hbm, adv_hbm, out_hbm,
             srcv, gsrcv, dstv, asv, adv, exbuf, coefbuf, denv, rows,
             zbuf, zv, dacc, nacc, sem):
        c = lax.axis_index("c")
        s = lax.axis_index("s")
        base = s * ENT

        pltpu.sync_copy(src_hbm.at[pl.ds(base, ENT)], srcv)
        pltpu.sync_copy(dst_hbm.at[pl.ds(base, ENT)], dstv)
        pltpu.sync_copy(asv_hbm, asv)
        pltpu.sync_copy(adv_hbm, adv)

        coff = jnp.full((16,), 0, jnp.int32) + c

        @pl.loop(0, ENT // 16, unroll=4)
        def _(g):
            sl = pl.ds(g * 16, 16)
            gsrcv[sl] = (srcv[sl] << 1) + coff

        gat = pltpu.async_copy(hs_hbm.at[gsrcv], rows, sem)

        @pl.loop(0, NPT * (HALF // 16))
        def _(i):
            zbuf[i // (HALF // 16), pl.ds((i % (HALF // 16)) * 16, 16)] = (
                jnp.zeros((16,), jnp.float32))

        @pl.loop(0, NPT // 16)
        def _(i):
            zv[pl.ds(i * 16, 16)] = jnp.zeros((16,), jnp.float32)

        pltpu.sync_copy(zv, dacc.at[pl.ds(s * NPT, NPT)])
        pltpu.sync_copy(zbuf, nacc.at[pl.ds(s * NPT, NPT)])
        plsc.subcore_barrier()

        @pl.loop(0, ENT // 16)
        def _(g):
            sl = pl.ds(g * 16, 16)
            e = plsc.load_gather(asv, [srcv[sl]]) + plsc.load_gather(adv, [dstv[sl]])
            e = jnp.where(e >= 0, e, 0.2 * e)
            exbuf[sl] = jnp.exp(e)

        pltpu.sync_copy(exbuf, dacc.at[dstv], add=True)
        plsc.subcore_barrier()
        pltpu.sync_copy(dacc, denv)

        @pl.loop(0, ENT // 16)
        def _(g):
            sl = pl.ds(g * 16, 16)
            den = plsc.load_gather(denv, [dstv[sl]])
            coefbuf[sl] = exbuf[sl] / (den + 1e-16)

        gat.wait()

        @pl.loop(0, ENT)
        def _(j):
            cf = plsc.load_gather(coefbuf, [jnp.full((16,), 0, jnp.int32) + j])
            for jj in range(HALF // 16):
                sl2 = pl.ds(jj * 16, 16)
                rows[j, sl2] = rows[j, sl2] * cf

        pltpu.sync_copy(rows, nacc.at[dstv], add=True)
        plsc.subcore_barrier()
        n0 = pl.multiple_of(s * NPT, NPT)
        pltpu.sync_copy(nacc.at[pl.ds(n0, NPT)],
                        out_hbm.at[pl.ds(n0, NPT), pl.ds(c * HALF, HALF)])

    return pl.kernel(body, out_type=out_type, mesh=_sc_mesh(),
                     scratch_types=scratch, compiler_params=_SC_PARAMS)


BN = 2776
BPB = BN // N_ROI


def _sage_dense_body(x_ref, agg_ref, deg_ref, wr_ref, wn_ref, b_ref,
                     out_ref, outq_ref):
    inv = 1.0 / jnp.maximum(deg_ref[...], 1.0)
    h = (jnp.dot(x_ref[...], wr_ref[...], preferred_element_type=jnp.float32)
         + jnp.dot(agg_ref[...] * inv, wn_ref[...],
                   preferred_element_type=jnp.float32)
         + b_ref[...])
    h = jax.nn.gelu(h)
    out_ref[...] = h
    for q in range(NQ):
        outq_ref[q] = h[:, QW * q:QW * (q + 1)]


def _sage_dense(xin, agg, degc, Wr, Wn, brow):
    return pl.pallas_call(
        _sage_dense_body,
        grid=(N_TOT // BN,),
        in_specs=[
            pl.BlockSpec((BN, D), lambda i: (i, 0)),
            pl.BlockSpec((BN, D), lambda i: (i, 0)),
            pl.BlockSpec((BN, 1), lambda i: (i, 0)),
            pl.BlockSpec((D, H), lambda i: (0, 0)),
            pl.BlockSpec((D, H), lambda i: (0, 0)),
            pl.BlockSpec((1, H), lambda i: (0, 0)),
        ],
        out_specs=[
            pl.BlockSpec((BN, D), lambda i: (i, 0)),
            pl.BlockSpec((NQ, BN, QW), lambda i: (0, i, 0)),
        ],
        out_shape=[
            jax.ShapeDtypeStruct((N_TOT, D), jnp.float32),
            jax.ShapeDtypeStruct((NQ, N_TOT, QW), jnp.float32),
        ],
    )(xin, agg, degc, Wr, Wn, brow)


def _sage_pool_body(x_ref, agg_ref, deg_ref, wr_ref, wn_ref, b_ref,
                    mask_ref, logit_ref, wg_ref, as_ref, ad_ref,
                    hp_ref, asv_ref, adv_ref, alpha_ref):
    @pl.when(pl.program_id(0) == 0)
    def _():
        logits = jnp.where(mask_ref[...] == 0, -20.0, logit_ref[...])
        m = jnp.max(logits, axis=0, keepdims=True)
        ex = jnp.exp(logits - m)
        alpha_ref[...] = ex / jnp.sum(ex, axis=0, keepdims=True)

    inv = 1.0 / jnp.maximum(deg_ref[...], 1.0)
    h = (jnp.dot(x_ref[...], wr_ref[...], preferred_element_type=jnp.float32)
         + jnp.dot(agg_ref[...] * inv, wn_ref[...],
                   preferred_element_type=jnp.float32)
         + b_ref[...])
    h = jax.nn.gelu(h)
    alpha = alpha_ref[...]
    zs = [lax.dot_general(alpha, h[N_ROI * b:N_ROI * (b + 1), :],
                          (((0,), (0,)), ((), ())),
                          preferred_element_type=jnp.float32)
          for b in range(BPB)]
    z = jnp.concatenate(zs, axis=0)
    hp = jnp.dot(z, wg_ref[...], preferred_element_type=jnp.float32)
    asv_ref[...] = jnp.dot(hp, as_ref[...], preferred_element_type=jnp.float32)
    adv_ref[...] = jnp.dot(hp, ad_ref[...], preferred_element_type=jnp.float32)
    hp_ref[...] = hp


def _sage_dense_pool(xin, agg, degc, Wr, Wn, brow, mask, pool_logits,
                     Wg0, as_col, ad_col):
    zrows = BPB * N_NET
    return pl.pallas_call(
        _sage_pool_body,
        grid=(N_TOT // BN,),
        in_specs=[
            pl.BlockSpec((BN, D), lambda i: (i, 0)),
            pl.BlockSpec((BN, D), lambda i: (i, 0)),
            pl.BlockSpec((BN, 1), lambda i: (i, 0)),
            pl.BlockSpec((D, H), lambda i: (0, 0)),
            pl.BlockSpec((D, H), lambda i: (0, 0)),
            pl.BlockSpec((1, H), lambda i: (0, 0)),
            pl.BlockSpec((N_ROI, N_NET), lambda i: (0, 0)),
            pl.BlockSpec((N_ROI, N_NET), lambda i: (0, 0)),
            pl.BlockSpec((D, H), lambda i: (0, 0)),
            pl.BlockSpec((H, 1), lambda i: (0, 0)),
            pl.BlockSpec((H, 1), lambda i: (0, 0)),
        ],
        out_specs=[
            pl.BlockSpec((zrows, D), lambda i: (i, 0)),
            pl.BlockSpec((zrows, 1), lambda i: (i, 0)),
            pl.BlockSpec((zrows, 1), lambda i: (i, 0)),
        ],
        out_shape=[
            jax.ShapeDtypeStruct((NN, D), jnp.float32),
            jax.ShapeDtypeStruct((NN, 1), jnp.float32),
            jax.ShapeDtypeStruct((NN, 1), jnp.float32),
        ],
        scratch_shapes=[pltpu.VMEM((N_ROI, N_NET), jnp.float32)],
    )(xin, agg, degc, Wr, Wn, brow, mask, pool_logits, Wg0, as_col, ad_col)


def _gat_mid_body(num_ref, b_ref, wg_ref, as_ref, ad_ref,
                  hp_ref, asv_ref, adv_ref):
    g = jax.nn.gelu(num_ref[...] + b_ref[...])
    h = jnp.dot(g, wg_ref[...], preferred_element_type=jnp.float32)
    asv_ref[...] = jnp.dot(h, as_ref[...], preferred_element_type=jnp.float32)
    adv_ref[...] = jnp.dot(h, ad_ref[...], preferred_element_type=jnp.float32)
    hp_ref[...] = h


def _gat_mid(num0, brow, Wg, as_col, ad_col):
    full = lambda s: pl.BlockSpec(s, lambda: tuple(0 for _ in s))
    return pl.pallas_call(
        _gat_mid_body,
        in_specs=[
            full((NN, D)),
            full((1, H)),
            full((D, H)),
            full((D, 1)),
            full((D, 1)),
        ],
        out_specs=[
            full((NN, D)),
            full((NN, 1)),
            full((NN, 1)),
        ],
        out_shape=[
            jax.ShapeDtypeStruct((NN, D), jnp.float32),
            jax.ShapeDtypeStruct((NN, 1), jnp.float32),
            jax.ShapeDtypeStruct((NN, 1), jnp.float32),
        ],
    )(num0, brow, Wg, as_col, ad_col)


def _head_body(num_ref, b_ref, wc_ref, bc_ref, wkf_ref, sel_ref, bk_ref, out_ref):
    g = jax.nn.gelu(num_ref[...] + b_ref[...])
    cmat = (jnp.dot(g, wc_ref[...], preferred_element_type=jnp.float32)
            + bc_ref[...])
    weighted = cmat * wkf_ref[...]
    out_ref[...] = jnp.dot(sel_ref[...], weighted,
                           preferred_element_type=jnp.float32) + bk_ref[0, 0]


def _head(num1, brow, Wc, bcrow, wkf, sel, bkr):
    full = lambda s: pl.BlockSpec(s, lambda: tuple(0 for _ in s))
    return pl.pallas_call(
        _head_body,
        in_specs=[
            full((NN, D)),
            full((1, D)),
            full((D, NC)),
            full((1, NC)),
            full((NN, NC)),
            full((B, NN)),
            full((1, 1)),
        ],
        out_specs=full((B, NC)),
        out_shape=jax.ShapeDtypeStruct((B, NC), jnp.float32),
    )(num1, brow, Wc, bcrow, wkf, sel, bkr)


_sage_sc_deg = _make_sage_sc(with_deg=True)
_sage_sc = _make_sage_sc(with_deg=False)
_gat_sc = _make_gat_sc()


def kernel(x, roi_edge_index, net_edge_index, mask, W_root0, W_neigh0, b0,
           W_root1, W_neigh1, b1, pool_logits, Wg0, att_s0, att_d0, bg0,
           Wg1, att_s1, att_d1, bg1, Wc, bc, Wk, bk):
    f32 = jnp.float32

    src = roi_edge_index[0].astype(jnp.int32)
    dst = roi_edge_index[1].astype(jnp.int32)
    pad = E_PAD - E_ROI
    src_p = jnp.concatenate([src, jnp.zeros((pad,), jnp.int32)])
    dst_p = jnp.concatenate([dst, jnp.full((pad,), N_TOT, jnp.int32)])
    pk = (src_p << 15) | dst_p
    pkf = lax.bitcast_convert_type(pk, jnp.float32)

    srcn = net_edge_index[0].astype(jnp.int32)
    dstn = net_edge_index[1].astype(jnp.int32)

    b0r = b0.reshape(1, H)
    b1r = b1.reshape(1, D)
    bg0r = bg0.reshape(1, H)
    bg1r = bg1.reshape(1, D)
    bcr = bc.reshape(1, NC)
    as0c = att_s0.reshape(H, 1)
    ad0c = att_d0.reshape(H, 1)
    as1c = att_s1.reshape(D, 1)
    ad1c = att_d1.reshape(D, 1)
    wkf = jnp.tile(Wk[:, 0], B).reshape(NN, 1) * jnp.ones((1, NC), f32)
    sel = jnp.repeat(jnp.eye(B, dtype=f32), N_NET, axis=1)

    xs0 = x.reshape(N_TOT, NQ, QW).transpose(1, 0, 2).reshape(NQ * N_TOT, QW)
    agg0, deg_pad = _sage_sc_deg(xs0, pkf)
    degc = deg_pad[:N_TOT].reshape(N_TOT, 1)
    h1, h1q = _sage_dense(x, agg0, degc, W_root0, W_neigh0, b0r)
    agg1 = _sage_sc(h1q.reshape(NQ * N_TOT, QW), pkf)

    hp0, as0, ad0 = _sage_dense_pool(h1, agg1, degc, W_root1,
                                     W_neigh1, b1r, mask, pool_logits,
                                     Wg0, as0c, ad0c)

    num0 = _gat_sc(hp0.reshape(NUM_CORES * NN, HALF), srcn, dstn,
                   as0.reshape(NN), ad0.reshape(NN))
    hp1, as1, ad1 = _gat_mid(num0, bg0r, Wg1, as1c, ad1c)
    num1 = _gat_sc(hp1.reshape(NUM_CORES * NN, HALF), srcn, dstn,
                   as1.reshape(NN), ad1.reshape(NN))

    return _head(num1, bg1r, Wc, bcr, wkf, sel, bk.reshape(1, 1))
